# baseline (speedup 1.0000x reference)
import functools

import jax
import jax.numpy as jnp
from jax import lax
from jax.experimental import pallas as pl
from jax.experimental.pallas import tpu as pltpu
from jax.experimental.pallas import tpu_sc as plsc

_NC = 2
_NS = 16
_NW = _NC * _NS

_SEQ = 200
_S0 = 96
_S1 = 104
_NSPLIT = 4


def _pool_sc(x, tslice, batch, ew):
    b_per_w = batch // _NW
    nch = ew // 16

    mesh = plsc.VectorSubcoreMesh(core_axis_name="c", subcore_axis_name="s")

    @functools.partial(
        pl.kernel,
        out_type=jax.ShapeDtypeStruct((batch, ew), jnp.float32),
        mesh=mesh,
        scratch_types=[
            pltpu.VMEM((b_per_w, _SEQ), jnp.int32),
            pltpu.VMEM((2, _SEQ, ew), jnp.float32),
            pltpu.VMEM((b_per_w, ew), jnp.float32),
            pltpu.SemaphoreType.DMA,
            pltpu.SemaphoreType.DMA,
        ],
        compiler_params=pltpu.CompilerParams(use_tc_tiling_on_sc=False),
    )
    def pool(x_hbm, tab_hbm, out_hbm, idx_v, rows_v, acc_v, sem0, sem1):
        wid = lax.axis_index("s") * _NC + lax.axis_index("c")
        pltpu.sync_copy(x_hbm.at[pl.ds(wid * b_per_w, b_per_w)], idx_v)

        def issue(r, buf, sem):
            pltpu.async_copy(
                tab_hbm.at[idx_v.at[r, pl.ds(0, _S0)]],
                rows_v.at[buf, pl.ds(0, _S0)], sem)
            pltpu.async_copy(
                tab_hbm.at[idx_v.at[r, pl.ds(_S0, _S1)]],
                rows_v.at[buf, pl.ds(_S0, _S1)], sem)

        def wait(buf, sem):
            pltpu.make_async_copy(
                tab_hbm.at[pl.ds(0, _SEQ)], rows_v.at[buf], sem).wait()

        def accumulate(r, buf):
            def body(s, carry):
                return tuple(
                    carry[j] + rows_v[buf, s, pl.ds(16 * j, 16)]
                    for j in range(nch))

            z = (jnp.zeros((16,), jnp.float32),) * nch
            acc = lax.fori_loop(0, _SEQ, body, z, unroll=8)
            for j in range(nch):
                acc_v[r, pl.ds(16 * j, 16)] = acc[j]

        issue(0, 0, sem0)

        def pair_body(g, _):
            r0 = 2 * g
            issue(r0 + 1, 1, sem1)
            wait(0, sem0)
            accumulate(r0, 0)

            @pl.when(r0 + 2 < b_per_w)
            def _issue_next():
                issue(r0 + 2, 0, sem0)

            wait(1, sem1)
            accumulate(r0 + 1, 1)
            return _

        lax.fori_loop(0, b_per_w // 2, pair_body, None)
        pltpu.sync_copy(acc_v, out_hbm.at[pl.ds(wid * b_per_w, b_per_w)])

    return pool(x, tslice)


def _mlp_body(p_ref, w1_ref, b1_ref, w2_ref, b2_ref, o_ref):
    p = p_ref[...] * (1.0 / _SEQ)
    h = jnp.dot(p, w1_ref[...], preferred_element_type=jnp.float32) + b1_ref[...]
    h = jnp.maximum(h, 0.0)
    logits = jnp.dot(h, w2_ref[...], preferred_element_type=jnp.float32) + b2_ref[...]
    m = jnp.max(logits, axis=1, keepdims=True)
    ex = jnp.exp(logits - m)
    o_ref[...] = logits - m - jnp.log(jnp.sum(ex, axis=1, keepdims=True))


def _mlp_tc(sums, W1, b1, W2, b2):
    batch, embed = sums.shape
    hidden = W1.shape[1]
    out = W2.shape[1]
    blk = 512
    return pl.pallas_call(
        _mlp_body,
        grid=(batch // blk,),
        in_specs=[
            pl.BlockSpec((blk, embed), lambda i: (i, 0)),
            pl.BlockSpec((embed, hidden), lambda i: (0, 0)),
            pl.BlockSpec((1, hidden), lambda i: (0, 0)),
            pl.BlockSpec((hidden, out), lambda i: (0, 0)),
            pl.BlockSpec((1, out), lambda i: (0, 0)),
        ],
        out_specs=pl.BlockSpec((blk, out), lambda i: (i, 0)),
        out_shape=jax.ShapeDtypeStruct((batch, out), jnp.float32),
    )(sums, W1, b1.reshape(1, hidden), W2, b2.reshape(1, out))


def kernel(x, table, W1, b1, W2, b2):
    batch, seq = x.shape
    embed = table.shape[1]
    assert seq == _SEQ and batch % _NW == 0 and embed % (16 * _NSPLIT) == 0
    ew = embed // _NSPLIT
    parts = [
        _pool_sc(x, lax.slice_in_dim(table, q * ew, (q + 1) * ew, axis=1),
                 batch, ew)
        for q in range(_NSPLIT)
    ]
    sums = jnp.concatenate(parts, axis=1)
    return _mlp_tc(sums, W1, b1, W2, b2)


# speedup vs baseline: 4.8870x; 4.8870x over previous
import functools

import jax
import jax.numpy as jnp
from jax import lax
from jax.experimental import pallas as pl
from jax.experimental.pallas import tpu as pltpu
from jax.experimental.pallas import tpu_sc as plsc

_NC = 2
_NS = 16
_NW = _NC * _NS

_SEQ = 200
_S0 = 96
_S1 = 104
_TBLK = 6400


def _stage_body(tt_ref, o_ref):
    o_ref[:, 0:64] = jnp.transpose(tt_ref[...], (1, 0))


def _stage_table(tt, vocab):
    return pl.pallas_call(
        _stage_body,
        grid=(pl.cdiv(vocab, _TBLK),),
        in_specs=[pl.BlockSpec((64, _TBLK), lambda i: (0, i))],
        out_specs=pl.BlockSpec((_TBLK, 128), lambda i: (i, 0)),
        out_shape=jax.ShapeDtypeStruct((vocab, 128), jnp.float32),
    )(tt)


def _pool_sc(x, tab, batch):
    b_per_w = batch // _NW

    mesh = plsc.VectorSubcoreMesh(core_axis_name="c", subcore_axis_name="s")

    @functools.partial(
        pl.kernel,
        out_type=jax.ShapeDtypeStruct((batch, 64), jnp.float32),
        mesh=mesh,
        scratch_types=[
            pltpu.VMEM((b_per_w, _SEQ), jnp.int32),
            pltpu.VMEM((2, _SEQ, 128), jnp.float32),
            pltpu.VMEM((b_per_w, 64), jnp.float32),
            pltpu.SemaphoreType.DMA,
            pltpu.SemaphoreType.DMA,
        ],
        compiler_params=pltpu.CompilerParams(use_tc_tiling_on_sc=False),
    )
    def pool(x_hbm, tab_hbm, out_hbm, idx_v, rows_v, acc_v, sem0, sem1):
        wid = lax.axis_index("s") * _NC + lax.axis_index("c")
        pltpu.sync_copy(x_hbm.at[pl.ds(wid * b_per_w, b_per_w)], idx_v)

        def issue(r, buf, sem):
            pltpu.async_copy(
                tab_hbm.at[idx_v.at[r, pl.ds(0, _S0)]],
                rows_v.at[buf, pl.ds(0, _S0)], sem)
            pltpu.async_copy(
                tab_hbm.at[idx_v.at[r, pl.ds(_S0, _S1)]],
                rows_v.at[buf, pl.ds(_S0, _S1)], sem)

        def wait(buf, sem):
            pltpu.make_async_copy(
                tab_hbm.at[pl.ds(0, _SEQ)], rows_v.at[buf], sem).wait()

        def accumulate(r, buf):
            def body(s, carry):
                a0, a1, a2, a3 = carry
                a0 = a0 + rows_v[buf, s, pl.ds(0, 16)]
                a1 = a1 + rows_v[buf, s, pl.ds(16, 16)]
                a2 = a2 + rows_v[buf, s, pl.ds(32, 16)]
                a3 = a3 + rows_v[buf, s, pl.ds(48, 16)]
                return a0, a1, a2, a3

            z = jnp.zeros((16,), jnp.float32)
            a0, a1, a2, a3 = lax.fori_loop(0, _SEQ, body, (z, z, z, z),
                                           unroll=8)
            acc_v[r, pl.ds(0, 16)] = a0
            acc_v[r, pl.ds(16, 16)] = a1
            acc_v[r, pl.ds(32, 16)] = a2
            acc_v[r, pl.ds(48, 16)] = a3

        issue(0, 0, sem0)

        def pair_body(g, _):
            r0 = 2 * g
            issue(r0 + 1, 1, sem1)
            wait(0, sem0)
            accumulate(r0, 0)

            @pl.when(r0 + 2 < b_per_w)
            def _issue_next():
                issue(r0 + 2, 0, sem0)

            wait(1, sem1)
            accumulate(r0 + 1, 1)
            return _

        lax.fori_loop(0, b_per_w // 2, pair_body, None)
        pltpu.sync_copy(acc_v, out_hbm.at[pl.ds(wid * b_per_w, b_per_w)])

    return pool(x, tab)


def _mlp_body(p_ref, w1_ref, b1_ref, w2_ref, b2_ref, o_ref):
    p = p_ref[...] * (1.0 / _SEQ)
    h = jnp.dot(p, w1_ref[...], preferred_element_type=jnp.float32) + b1_ref[...]
    h = jnp.maximum(h, 0.0)
    logits = jnp.dot(h, w2_ref[...], preferred_element_type=jnp.float32) + b2_ref[...]
    m = jnp.max(logits, axis=1, keepdims=True)
    ex = jnp.exp(logits - m)
    o_ref[...] = logits - m - jnp.log(jnp.sum(ex, axis=1, keepdims=True))


def _mlp_tc(sums, W1, b1, W2, b2):
    batch, embed = sums.shape
    hidden = W1.shape[1]
    out = W2.shape[1]
    blk = 512
    return pl.pallas_call(
        _mlp_body,
        grid=(batch // blk,),
        in_specs=[
            pl.BlockSpec((blk, embed), lambda i: (i, 0)),
            pl.BlockSpec((embed, hidden), lambda i: (0, 0)),
            pl.BlockSpec((1, hidden), lambda i: (0, 0)),
            pl.BlockSpec((hidden, out), lambda i: (0, 0)),
            pl.BlockSpec((1, out), lambda i: (0, 0)),
        ],
        out_specs=pl.BlockSpec((blk, out), lambda i: (i, 0)),
        out_shape=jax.ShapeDtypeStruct((batch, out), jnp.float32),
    )(sums, W1, b1.reshape(1, hidden), W2, b2.reshape(1, out))


def kernel(x, table, W1, b1, W2, b2):
    batch, seq = x.shape
    vocab, embed = table.shape
    assert seq == _SEQ and batch % _NW == 0 and embed == 64
    tab = _stage_table(table.T, vocab)
    sums = _pool_sc(x, tab, batch)
    return _mlp_tc(sums, W1, b1, W2, b2)


# speedup vs baseline: 5.7606x; 1.1788x over previous
import functools

import jax
import jax.numpy as jnp
from jax import lax
from jax.experimental import pallas as pl
from jax.experimental.pallas import tpu as pltpu
from jax.experimental.pallas import tpu_sc as plsc

_NC = 2
_NS = 16
_NW = _NC * _NS

_SEQ = 200
_S0 = 96
_S1 = 104
_TBLK = 6400


def _stage_body(tt_ref, o_ref):
    o_ref[:, 0:64] = jnp.transpose(tt_ref[...], (1, 0))


def _stage_table(tt, vocab):
    return pl.pallas_call(
        _stage_body,
        grid=(pl.cdiv(vocab, _TBLK),),
        in_specs=[pl.BlockSpec((64, _TBLK), lambda i: (0, i))],
        out_specs=pl.BlockSpec((_TBLK, 128), lambda i: (i, 0)),
        out_shape=jax.ShapeDtypeStruct((vocab, 128), jnp.float32),
    )(tt)


def _pool_sc(x, tab, batch):
    b_per_w = batch // _NW

    mesh = plsc.VectorSubcoreMesh(core_axis_name="c", subcore_axis_name="s")

    @functools.partial(
        pl.kernel,
        out_type=jax.ShapeDtypeStruct((batch, 64), jnp.float32),
        mesh=mesh,
        scratch_types=[
            pltpu.VMEM((b_per_w, _SEQ), jnp.int32),
            pltpu.VMEM((2, _SEQ, 64), jnp.float32),
            pltpu.VMEM((b_per_w, 64), jnp.float32),
            pltpu.SemaphoreType.DMA,
            pltpu.SemaphoreType.DMA,
        ],
        compiler_params=pltpu.CompilerParams(use_tc_tiling_on_sc=False),
    )
    def pool(x_hbm, tab_hbm, out_hbm, idx_v, rows_v, acc_v, sem0, sem1):
        wid = lax.axis_index("s") * _NC + lax.axis_index("c")
        pltpu.sync_copy(x_hbm.at[pl.ds(wid * b_per_w, b_per_w)], idx_v)

        def issue(r, buf, sem):
            pltpu.async_copy(
                tab_hbm.at[idx_v.at[r, pl.ds(0, _S0)]],
                rows_v.at[buf, pl.ds(0, _S0)], sem)
            pltpu.async_copy(
                tab_hbm.at[idx_v.at[r, pl.ds(_S0, _S1)]],
                rows_v.at[buf, pl.ds(_S0, _S1)], sem)

        def wait(buf, sem):
            pltpu.make_async_copy(
                tab_hbm.at[pl.ds(0, _SEQ)], rows_v.at[buf], sem).wait()

        def accumulate(r, buf):
            def body(s, carry):
                a0, a1, a2, a3 = carry
                a0 = a0 + rows_v[buf, s, pl.ds(0, 16)]
                a1 = a1 + rows_v[buf, s, pl.ds(16, 16)]
                a2 = a2 + rows_v[buf, s, pl.ds(32, 16)]
                a3 = a3 + rows_v[buf, s, pl.ds(48, 16)]
                return a0, a1, a2, a3

            z = jnp.zeros((16,), jnp.float32)
            a0, a1, a2, a3 = lax.fori_loop(0, _SEQ, body, (z, z, z, z),
                                           unroll=8)
            acc_v[r, pl.ds(0, 16)] = a0
            acc_v[r, pl.ds(16, 16)] = a1
            acc_v[r, pl.ds(32, 16)] = a2
            acc_v[r, pl.ds(48, 16)] = a3

        issue(0, 0, sem0)

        def pair_body(g, _):
            r0 = 2 * g
            issue(r0 + 1, 1, sem1)
            wait(0, sem0)
            accumulate(r0, 0)

            @pl.when(r0 + 2 < b_per_w)
            def _issue_next():
                issue(r0 + 2, 0, sem0)

            wait(1, sem1)
            accumulate(r0 + 1, 1)
            return _

        lax.fori_loop(0, b_per_w // 2, pair_body, None)
        pltpu.sync_copy(acc_v, out_hbm.at[pl.ds(wid * b_per_w, b_per_w)])

    return pool(x, tab)


def _mlp_body(p_ref, w1_ref, b1_ref, w2_ref, b2_ref, o_ref):
    p = p_ref[...] * (1.0 / _SEQ)
    h = jnp.dot(p, w1_ref[...], preferred_element_type=jnp.float32) + b1_ref[...]
    h = jnp.maximum(h, 0.0)
    logits = jnp.dot(h, w2_ref[...], preferred_element_type=jnp.float32) + b2_ref[...]
    m = jnp.max(logits, axis=1, keepdims=True)
    ex = jnp.exp(logits - m)
    o_ref[...] = logits - m - jnp.log(jnp.sum(ex, axis=1, keepdims=True))


def _mlp_tc(sums, W1, b1, W2, b2):
    batch, embed = sums.shape
    hidden = W1.shape[1]
    out = W2.shape[1]
    blk = 512
    return pl.pallas_call(
        _mlp_body,
        grid=(batch // blk,),
        in_specs=[
            pl.BlockSpec((blk, embed), lambda i: (i, 0)),
            pl.BlockSpec((embed, hidden), lambda i: (0, 0)),
            pl.BlockSpec((1, hidden), lambda i: (0, 0)),
            pl.BlockSpec((hidden, out), lambda i: (0, 0)),
            pl.BlockSpec((1, out), lambda i: (0, 0)),
        ],
        out_specs=pl.BlockSpec((blk, out), lambda i: (i, 0)),
        out_shape=jax.ShapeDtypeStruct((batch, out), jnp.float32),
    )(sums, W1, b1.reshape(1, hidden), W2, b2.reshape(1, out))


def kernel(x, table, W1, b1, W2, b2):
    batch, seq = x.shape
    vocab, embed = table.shape
    assert seq == _SEQ and batch % _NW == 0 and embed == 64
    tab = _stage_table(table.T, vocab).reshape(2 * vocab, 64)
    sums = _pool_sc(x * 2, tab, batch)
    return _mlp_tc(sums, W1, b1, W2, b2)


# speedup vs baseline: 5.8139x; 1.0092x over previous
import functools

import jax
import jax.numpy as jnp
from jax import lax
from jax.experimental import pallas as pl
from jax.experimental.pallas import tpu as pltpu
from jax.experimental.pallas import tpu_sc as plsc

_NC = 2
_NS = 16
_NW = _NC * _NS

_SEQ = 200
_S0 = 96
_S1 = 104
_TBLK = 6400


def _stage_body(tt_ref, o_ref):
    h = _TBLK // 2
    o_ref[:, 0:64] = jnp.transpose(tt_ref[:, 0:h], (1, 0))
    o_ref[:, 64:128] = jnp.transpose(tt_ref[:, h:_TBLK], (1, 0))


def _stage_table(tt, vocab):
    return pl.pallas_call(
        _stage_body,
        grid=(pl.cdiv(vocab, _TBLK),),
        in_specs=[pl.BlockSpec((64, _TBLK), lambda i: (0, i))],
        out_specs=pl.BlockSpec((_TBLK // 2, 128), lambda i: (i, 0)),
        out_shape=jax.ShapeDtypeStruct(
            (pl.cdiv(vocab, _TBLK) * (_TBLK // 2), 128), jnp.float32),
    )(tt)


def _pool_sc(x, tab, batch):
    b_per_w = batch // _NW

    mesh = plsc.VectorSubcoreMesh(core_axis_name="c", subcore_axis_name="s")

    @functools.partial(
        pl.kernel,
        out_type=jax.ShapeDtypeStruct((batch, 64), jnp.float32),
        mesh=mesh,
        scratch_types=[
            pltpu.VMEM((b_per_w, _SEQ), jnp.int32),
            pltpu.VMEM((2, _SEQ, 64), jnp.float32),
            pltpu.VMEM((b_per_w, 64), jnp.float32),
            pltpu.SemaphoreType.DMA,
            pltpu.SemaphoreType.DMA,
        ],
        compiler_params=pltpu.CompilerParams(use_tc_tiling_on_sc=False),
    )
    def pool(x_hbm, tab_hbm, out_hbm, idx_v, rows_v, acc_v, sem0, sem1):
        wid = lax.axis_index("s") * _NC + lax.axis_index("c")
        pltpu.sync_copy(x_hbm.at[pl.ds(wid * b_per_w, b_per_w)], idx_v)

        def issue(r, buf, sem):
            pltpu.async_copy(
                tab_hbm.at[idx_v.at[r, pl.ds(0, _S0)]],
                rows_v.at[buf, pl.ds(0, _S0)], sem)
            pltpu.async_copy(
                tab_hbm.at[idx_v.at[r, pl.ds(_S0, _S1)]],
                rows_v.at[buf, pl.ds(_S0, _S1)], sem)

        def wait(buf, sem):
            pltpu.make_async_copy(
                tab_hbm.at[pl.ds(0, _SEQ)], rows_v.at[buf], sem).wait()

        def accumulate(r, buf):
            def body(s, carry):
                a0, a1, a2, a3 = carry
                a0 = a0 + rows_v[buf, s, pl.ds(0, 16)]
                a1 = a1 + rows_v[buf, s, pl.ds(16, 16)]
                a2 = a2 + rows_v[buf, s, pl.ds(32, 16)]
                a3 = a3 + rows_v[buf, s, pl.ds(48, 16)]
                return a0, a1, a2, a3

            z = jnp.zeros((16,), jnp.float32)
            a0, a1, a2, a3 = lax.fori_loop(0, _SEQ, body, (z, z, z, z),
                                           unroll=8)
            acc_v[r, pl.ds(0, 16)] = a0
            acc_v[r, pl.ds(16, 16)] = a1
            acc_v[r, pl.ds(32, 16)] = a2
            acc_v[r, pl.ds(48, 16)] = a3

        issue(0, 0, sem0)

        def pair_body(g, _):
            r0 = 2 * g
            issue(r0 + 1, 1, sem1)
            wait(0, sem0)
            accumulate(r0, 0)

            @pl.when(r0 + 2 < b_per_w)
            def _issue_next():
                issue(r0 + 2, 0, sem0)

            wait(1, sem1)
            accumulate(r0 + 1, 1)
            return _

        lax.fori_loop(0, b_per_w // 2, pair_body, None)
        pltpu.sync_copy(acc_v, out_hbm.at[pl.ds(wid * b_per_w, b_per_w)])

    return pool(x, tab)


def _mlp_body(p_ref, w1_ref, b1_ref, w2_ref, b2_ref, o_ref):
    p = p_ref[...] * (1.0 / _SEQ)
    h = jnp.dot(p, w1_ref[...], preferred_element_type=jnp.float32) + b1_ref[...]
    h = jnp.maximum(h, 0.0)
    logits = jnp.dot(h, w2_ref[...], preferred_element_type=jnp.float32) + b2_ref[...]
    m = jnp.max(logits, axis=1, keepdims=True)
    ex = jnp.exp(logits - m)
    o_ref[...] = logits - m - jnp.log(jnp.sum(ex, axis=1, keepdims=True))


def _mlp_tc(sums, W1, b1, W2, b2):
    batch, embed = sums.shape
    hidden = W1.shape[1]
    out = W2.shape[1]
    blk = 512
    return pl.pallas_call(
        _mlp_body,
        grid=(batch // blk,),
        in_specs=[
            pl.BlockSpec((blk, embed), lambda i: (i, 0)),
            pl.BlockSpec((embed, hidden), lambda i: (0, 0)),
            pl.BlockSpec((1, hidden), lambda i: (0, 0)),
            pl.BlockSpec((hidden, out), lambda i: (0, 0)),
            pl.BlockSpec((1, out), lambda i: (0, 0)),
        ],
        out_specs=pl.BlockSpec((blk, out), lambda i: (i, 0)),
        out_shape=jax.ShapeDtypeStruct((batch, out), jnp.float32),
    )(sums, W1, b1.reshape(1, hidden), W2, b2.reshape(1, out))


def kernel(x, table, W1, b1, W2, b2):
    batch, seq = x.shape
    vocab, embed = table.shape
    assert seq == _SEQ and batch % _NW == 0 and embed == 64
    nrow = 2 * pl.cdiv(vocab, _TBLK) * (_TBLK // 2)
    tab = _stage_table(table.T, vocab).reshape(nrow, 64)
    u = x % _TBLK
    xf = x - u + 2 * (u % (_TBLK // 2)) + u // (_TBLK // 2)
    sums = _pool_sc(xf, tab, batch)
    return _mlp_tc(sums, W1, b1, W2, b2)


# speedup vs baseline: 6.3943x; 1.0998x over previous
import functools

import jax
import jax.numpy as jnp
from jax import lax
from jax.experimental import pallas as pl
from jax.experimental.pallas import tpu as pltpu
from jax.experimental.pallas import tpu_sc as plsc

_NC = 2
_NS = 16
_NW = _NC * _NS

_SEQ = 200
_S0 = 96
_S1 = 104
_TBLK = 6400


def _stage_body(tt_ref, o_ref):
    h = _TBLK // 2
    o_ref[:, 0:64] = jnp.transpose(tt_ref[:, 0:h], (1, 0))
    o_ref[:, 64:128] = jnp.transpose(tt_ref[:, h:_TBLK], (1, 0))


def _stage_table(tt, vocab):
    return pl.pallas_call(
        _stage_body,
        grid=(pl.cdiv(vocab, _TBLK),),
        in_specs=[pl.BlockSpec((64, _TBLK), lambda i: (0, i))],
        out_specs=pl.BlockSpec((_TBLK // 2, 128), lambda i: (i, 0)),
        out_shape=jax.ShapeDtypeStruct(
            (pl.cdiv(vocab, _TBLK) * (_TBLK // 2), 128), jnp.float32),
    )(tt)


def _pool_sc(x, tab, batch):
    b_per_w = batch // _NW

    mesh = plsc.VectorSubcoreMesh(core_axis_name="c", subcore_axis_name="s")

    @functools.partial(
        pl.kernel,
        out_type=jax.ShapeDtypeStruct((batch, 64), jnp.float32),
        mesh=mesh,
        scratch_types=[
            pltpu.VMEM((b_per_w, _SEQ), jnp.int32),
            pltpu.VMEM((4, _SEQ, 64), jnp.float32),
            pltpu.VMEM((b_per_w, 64), jnp.float32),
            pltpu.SemaphoreType.DMA,
            pltpu.SemaphoreType.DMA,
            pltpu.SemaphoreType.DMA,
            pltpu.SemaphoreType.DMA,
        ],
        compiler_params=pltpu.CompilerParams(use_tc_tiling_on_sc=False),
    )
    def pool(x_hbm, tab_hbm, out_hbm, idx_v, rows_v, acc_v,
             sem0, sem1, sem2, sem3):
        sems = (sem0, sem1, sem2, sem3)
        wid = lax.axis_index("s") * _NC + lax.axis_index("c")
        pltpu.sync_copy(x_hbm.at[pl.ds(wid * b_per_w, b_per_w)], idx_v)

        def issue(r, buf, sem):
            pltpu.async_copy(
                tab_hbm.at[idx_v.at[r, pl.ds(0, _S0)]],
                rows_v.at[buf, pl.ds(0, _S0)], sem)
            pltpu.async_copy(
                tab_hbm.at[idx_v.at[r, pl.ds(_S0, _S1)]],
                rows_v.at[buf, pl.ds(_S0, _S1)], sem)

        def wait(buf, sem):
            pltpu.make_async_copy(
                tab_hbm.at[pl.ds(0, _SEQ)], rows_v.at[buf], sem).wait()

        def accumulate(r, buf):
            def body(s, carry):
                a0, a1, a2, a3 = carry
                a0 = a0 + rows_v[buf, s, pl.ds(0, 16)]
                a1 = a1 + rows_v[buf, s, pl.ds(16, 16)]
                a2 = a2 + rows_v[buf, s, pl.ds(32, 16)]
                a3 = a3 + rows_v[buf, s, pl.ds(48, 16)]
                return a0, a1, a2, a3

            z = jnp.zeros((16,), jnp.float32)
            a0, a1, a2, a3 = lax.fori_loop(0, _SEQ, body, (z, z, z, z),
                                           unroll=8)
            acc_v[r, pl.ds(0, 16)] = a0
            acc_v[r, pl.ds(16, 16)] = a1
            acc_v[r, pl.ds(32, 16)] = a2
            acc_v[r, pl.ds(48, 16)] = a3

        issue(0, 0, sem0)
        issue(1, 1, sem1)
        issue(2, 2, sem2)

        def quad_body(g, _):
            r0 = 4 * g
            for k in range(4):
                nxt = r0 + k + 3

                @pl.when(nxt < b_per_w)
                def _issue_next():
                    issue(nxt, (k + 3) % 4, sems[(k + 3) % 4])

                wait(k, sems[k])
                accumulate(r0 + k, k)
            return _

        lax.fori_loop(0, b_per_w // 4, quad_body, None)
        pltpu.sync_copy(acc_v, out_hbm.at[pl.ds(wid * b_per_w, b_per_w)])

    return pool(x, tab)


def _mlp_body(p_ref, w1_ref, b1_ref, w2_ref, b2_ref, o_ref):
    p = p_ref[...] * (1.0 / _SEQ)
    h = jnp.dot(p, w1_ref[...], preferred_element_type=jnp.float32) + b1_ref[...]
    h = jnp.maximum(h, 0.0)
    logits = jnp.dot(h, w2_ref[...], preferred_element_type=jnp.float32) + b2_ref[...]
    m = jnp.max(logits, axis=1, keepdims=True)
    ex = jnp.exp(logits - m)
    o_ref[...] = logits - m - jnp.log(jnp.sum(ex, axis=1, keepdims=True))


def _mlp_tc(sums, W1, b1, W2, b2):
    batch, embed = sums.shape
    hidden = W1.shape[1]
    out = W2.shape[1]
    blk = 512
    return pl.pallas_call(
        _mlp_body,
        grid=(batch // blk,),
        in_specs=[
            pl.BlockSpec((blk, embed), lambda i: (i, 0)),
            pl.BlockSpec((embed, hidden), lambda i: (0, 0)),
            pl.BlockSpec((1, hidden), lambda i: (0, 0)),
            pl.BlockSpec((hidden, out), lambda i: (0, 0)),
            pl.BlockSpec((1, out), lambda i: (0, 0)),
        ],
        out_specs=pl.BlockSpec((blk, out), lambda i: (i, 0)),
        out_shape=jax.ShapeDtypeStruct((batch, out), jnp.float32),
    )(sums, W1, b1.reshape(1, hidden), W2, b2.reshape(1, out))


def kernel(x, table, W1, b1, W2, b2):
    batch, seq = x.shape
    vocab, embed = table.shape
    assert seq == _SEQ and batch % _NW == 0 and embed == 64
    nrow = 2 * pl.cdiv(vocab, _TBLK) * (_TBLK // 2)
    tab = _stage_table(table.T, vocab).reshape(nrow, 64)
    u = x % _TBLK
    xf = x - u + 2 * (u % (_TBLK // 2)) + u // (_TBLK // 2)
    sums = _pool_sc(xf, tab, batch)
    return _mlp_tc(sums, W1, b1, W2, b2)


# speedup vs baseline: 7.1610x; 1.1199x over previous
import functools

import jax
import jax.numpy as jnp
from jax import lax
from jax.experimental import pallas as pl
from jax.experimental.pallas import tpu as pltpu
from jax.experimental.pallas import tpu_sc as plsc

_NC = 2
_NS = 16
_NW = _NC * _NS

_SEQ = 200
_S0 = 96
_S1 = 104
_TBLK = 12800


def _stage_body(tt_ref, o_ref):
    h = _TBLK // 2
    o_ref[:, 0:64] = jnp.transpose(tt_ref[:, 0:h], (1, 0))
    o_ref[:, 64:128] = jnp.transpose(tt_ref[:, h:_TBLK], (1, 0))


def _stage_table(tt, vocab):
    return pl.pallas_call(
        _stage_body,
        grid=(pl.cdiv(vocab, _TBLK),),
        in_specs=[pl.BlockSpec((64, _TBLK), lambda i: (0, i))],
        out_specs=pl.BlockSpec((_TBLK // 2, 128), lambda i: (i, 0)),
        out_shape=jax.ShapeDtypeStruct(
            (pl.cdiv(vocab, _TBLK) * (_TBLK // 2), 128), jnp.float32),
    )(tt)


def _pool_sc(x, tab, batch):
    b_per_w = batch // _NW

    mesh = plsc.VectorSubcoreMesh(core_axis_name="c", subcore_axis_name="s")

    @functools.partial(
        pl.kernel,
        out_type=jax.ShapeDtypeStruct((batch, 64), jnp.float32),
        mesh=mesh,
        scratch_types=[
            pltpu.VMEM((b_per_w, _SEQ), jnp.int32),
            pltpu.VMEM((4, _SEQ, 64), jnp.float32),
            pltpu.VMEM((b_per_w, 64), jnp.float32),
            pltpu.SemaphoreType.DMA,
            pltpu.SemaphoreType.DMA,
            pltpu.SemaphoreType.DMA,
            pltpu.SemaphoreType.DMA,
        ],
        compiler_params=pltpu.CompilerParams(use_tc_tiling_on_sc=False),
    )
    def pool(x_hbm, tab_hbm, out_hbm, idx_v, rows_v, acc_v,
             sem0, sem1, sem2, sem3):
        sems = (sem0, sem1, sem2, sem3)
        wid = lax.axis_index("s") * _NC + lax.axis_index("c")
        pltpu.sync_copy(x_hbm.at[pl.ds(wid * b_per_w, b_per_w)], idx_v)

        def issue(r, buf, sem):
            pltpu.async_copy(
                tab_hbm.at[idx_v.at[r, pl.ds(0, _S0)]],
                rows_v.at[buf, pl.ds(0, _S0)], sem)
            pltpu.async_copy(
                tab_hbm.at[idx_v.at[r, pl.ds(_S0, _S1)]],
                rows_v.at[buf, pl.ds(_S0, _S1)], sem)

        def wait(buf, sem):
            pltpu.make_async_copy(
                tab_hbm.at[pl.ds(0, _SEQ)], rows_v.at[buf], sem).wait()

        def accumulate(r, buf):
            def body(s, carry):
                a0, a1, a2, a3 = carry
                a0 = a0 + rows_v[buf, s, pl.ds(0, 16)]
                a1 = a1 + rows_v[buf, s, pl.ds(16, 16)]
                a2 = a2 + rows_v[buf, s, pl.ds(32, 16)]
                a3 = a3 + rows_v[buf, s, pl.ds(48, 16)]
                return a0, a1, a2, a3

            z = jnp.zeros((16,), jnp.float32)
            a0, a1, a2, a3 = lax.fori_loop(0, _SEQ, body, (z, z, z, z),
                                           unroll=8)
            acc_v[r, pl.ds(0, 16)] = a0
            acc_v[r, pl.ds(16, 16)] = a1
            acc_v[r, pl.ds(32, 16)] = a2
            acc_v[r, pl.ds(48, 16)] = a3

        issue(0, 0, sem0)
        issue(1, 1, sem1)
        issue(2, 2, sem2)

        def quad_body(g, _):
            r0 = 4 * g
            for k in range(4):
                nxt = r0 + k + 3

                @pl.when(nxt < b_per_w)
                def _issue_next():
                    issue(nxt, (k + 3) % 4, sems[(k + 3) % 4])

                wait(k, sems[k])
                accumulate(r0 + k, k)
            return _

        lax.fori_loop(0, b_per_w // 4, quad_body, None)
        pltpu.sync_copy(acc_v, out_hbm.at[pl.ds(wid * b_per_w, b_per_w)])

    return pool(x, tab)


def _mlp_body(p_ref, w1_ref, b1_ref, w2_ref, b2_ref, o_ref):
    p = p_ref[...] * (1.0 / _SEQ)
    h = jnp.dot(p, w1_ref[...], preferred_element_type=jnp.float32) + b1_ref[...]
    h = jnp.maximum(h, 0.0)
    logits = jnp.dot(h, w2_ref[...], preferred_element_type=jnp.float32) + b2_ref[...]
    m = jnp.max(logits, axis=1, keepdims=True)
    ex = jnp.exp(logits - m)
    o_ref[...] = logits - m - jnp.log(jnp.sum(ex, axis=1, keepdims=True))


def _mlp_tc(sums, W1, b1, W2, b2):
    batch, embed = sums.shape
    hidden = W1.shape[1]
    out = W2.shape[1]
    blk = 512
    return pl.pallas_call(
        _mlp_body,
        grid=(batch // blk,),
        in_specs=[
            pl.BlockSpec((blk, embed), lambda i: (i, 0)),
            pl.BlockSpec((embed, hidden), lambda i: (0, 0)),
            pl.BlockSpec((1, hidden), lambda i: (0, 0)),
            pl.BlockSpec((hidden, out), lambda i: (0, 0)),
            pl.BlockSpec((1, out), lambda i: (0, 0)),
        ],
        out_specs=pl.BlockSpec((blk, out), lambda i: (i, 0)),
        out_shape=jax.ShapeDtypeStruct((batch, out), jnp.float32),
    )(sums, W1, b1.reshape(1, hidden), W2, b2.reshape(1, out))


def kernel(x, table, W1, b1, W2, b2):
    batch, seq = x.shape
    vocab, embed = table.shape
    assert seq == _SEQ and batch % _NW == 0 and embed == 64
    nrow = 2 * pl.cdiv(vocab, _TBLK) * (_TBLK // 2)
    tab = _stage_table(table.T, vocab).reshape(nrow, 64)
    u = x % _TBLK
    xf = x - u + 2 * (u % (_TBLK // 2)) + u // (_TBLK // 2)
    sums = _pool_sc(xf, tab, batch)
    return _mlp_tc(sums, W1, b1, W2, b2)


# speedup vs baseline: 7.5501x; 1.0543x over previous
import functools

import jax
import jax.numpy as jnp
from jax import lax
from jax.experimental import pallas as pl
from jax.experimental.pallas import tpu as pltpu
from jax.experimental.pallas import tpu_sc as plsc

_NC = 2
_NS = 16
_NW = _NC * _NS

_SEQ = 200
_S0 = 96
_S1 = 104
_TBLK = 25600


def _stage_body(tt_ref, o_ref):
    h = _TBLK // 2
    o_ref[:, 0:64] = jnp.transpose(tt_ref[:, 0:h], (1, 0))
    o_ref[:, 64:128] = jnp.transpose(tt_ref[:, h:_TBLK], (1, 0))


def _stage_table(tt, vocab):
    return pl.pallas_call(
        _stage_body,
        grid=(pl.cdiv(vocab, _TBLK),),
        in_specs=[pl.BlockSpec((64, _TBLK), lambda i: (0, i))],
        out_specs=pl.BlockSpec((_TBLK // 2, 128), lambda i: (i, 0)),
        out_shape=jax.ShapeDtypeStruct(
            (pl.cdiv(vocab, _TBLK) * (_TBLK // 2), 128), jnp.float32),
    )(tt)


def _pool_sc(x, tab, batch):
    b_per_w = batch // _NW

    mesh = plsc.VectorSubcoreMesh(core_axis_name="c", subcore_axis_name="s")

    @functools.partial(
        pl.kernel,
        out_type=jax.ShapeDtypeStruct((batch, 64), jnp.float32),
        mesh=mesh,
        scratch_types=[
            pltpu.VMEM((b_per_w, _SEQ), jnp.int32),
            pltpu.VMEM((4, _SEQ, 64), jnp.float32),
            pltpu.VMEM((b_per_w, 64), jnp.float32),
            pltpu.SemaphoreType.DMA,
            pltpu.SemaphoreType.DMA,
            pltpu.SemaphoreType.DMA,
            pltpu.SemaphoreType.DMA,
        ],
        compiler_params=pltpu.CompilerParams(use_tc_tiling_on_sc=False),
    )
    def pool(x_hbm, tab_hbm, out_hbm, idx_v, rows_v, acc_v,
             sem0, sem1, sem2, sem3):
        sems = (sem0, sem1, sem2, sem3)
        wid = lax.axis_index("s") * _NC + lax.axis_index("c")
        pltpu.sync_copy(x_hbm.at[pl.ds(wid * b_per_w, b_per_w)], idx_v)

        def issue(r, buf, sem):
            pltpu.async_copy(
                tab_hbm.at[idx_v.at[r, pl.ds(0, _S0)]],
                rows_v.at[buf, pl.ds(0, _S0)], sem)
            pltpu.async_copy(
                tab_hbm.at[idx_v.at[r, pl.ds(_S0, _S1)]],
                rows_v.at[buf, pl.ds(_S0, _S1)], sem)

        def wait(buf, sem):
            pltpu.make_async_copy(
                tab_hbm.at[pl.ds(0, _SEQ)], rows_v.at[buf], sem).wait()

        def accumulate(r, buf):
            def body(s, carry):
                a0, a1, a2, a3 = carry
                a0 = a0 + rows_v[buf, s, pl.ds(0, 16)]
                a1 = a1 + rows_v[buf, s, pl.ds(16, 16)]
                a2 = a2 + rows_v[buf, s, pl.ds(32, 16)]
                a3 = a3 + rows_v[buf, s, pl.ds(48, 16)]
                return a0, a1, a2, a3

            z = jnp.zeros((16,), jnp.float32)
            a0, a1, a2, a3 = lax.fori_loop(0, _SEQ, body, (z, z, z, z),
                                           unroll=8)
            acc_v[r, pl.ds(0, 16)] = a0
            acc_v[r, pl.ds(16, 16)] = a1
            acc_v[r, pl.ds(32, 16)] = a2
            acc_v[r, pl.ds(48, 16)] = a3

        issue(0, 0, sem0)
        issue(1, 1, sem1)
        issue(2, 2, sem2)

        def quad_body(g, _):
            r0 = 4 * g
            for k in range(4):
                nxt = r0 + k + 3

                @pl.when(nxt < b_per_w)
                def _issue_next():
                    issue(nxt, (k + 3) % 4, sems[(k + 3) % 4])

                wait(k, sems[k])
                accumulate(r0 + k, k)
            return _

        lax.fori_loop(0, b_per_w // 4, quad_body, None)
        pltpu.sync_copy(acc_v, out_hbm.at[pl.ds(wid * b_per_w, b_per_w)])

    return pool(x, tab)


def _mlp_body(p_ref, w1_ref, b1_ref, w2_ref, b2_ref, o_ref):
    p = p_ref[...] * (1.0 / _SEQ)
    h = jnp.dot(p, w1_ref[...], preferred_element_type=jnp.float32) + b1_ref[...]
    h = jnp.maximum(h, 0.0)
    logits = jnp.dot(h, w2_ref[...], preferred_element_type=jnp.float32) + b2_ref[...]
    m = jnp.max(logits, axis=1, keepdims=True)
    ex = jnp.exp(logits - m)
    o_ref[...] = logits - m - jnp.log(jnp.sum(ex, axis=1, keepdims=True))


def _mlp_tc(sums, W1, b1, W2, b2):
    batch, embed = sums.shape
    hidden = W1.shape[1]
    out = W2.shape[1]
    blk = 512
    return pl.pallas_call(
        _mlp_body,
        grid=(batch // blk,),
        in_specs=[
            pl.BlockSpec((blk, embed), lambda i: (i, 0)),
            pl.BlockSpec((embed, hidden), lambda i: (0, 0)),
            pl.BlockSpec((1, hidden), lambda i: (0, 0)),
            pl.BlockSpec((hidden, out), lambda i: (0, 0)),
            pl.BlockSpec((1, out), lambda i: (0, 0)),
        ],
        out_specs=pl.BlockSpec((blk, out), lambda i: (i, 0)),
        out_shape=jax.ShapeDtypeStruct((batch, out), jnp.float32),
    )(sums, W1, b1.reshape(1, hidden), W2, b2.reshape(1, out))


def kernel(x, table, W1, b1, W2, b2):
    batch, seq = x.shape
    vocab, embed = table.shape
    assert seq == _SEQ and batch % _NW == 0 and embed == 64
    nrow = 2 * pl.cdiv(vocab, _TBLK) * (_TBLK // 2)
    tab = _stage_table(table.T, vocab).reshape(nrow, 64)
    u = x % _TBLK
    xf = x - u + 2 * (u % (_TBLK // 2)) + u // (_TBLK // 2)
    sums = _pool_sc(xf, tab, batch)
    return _mlp_tc(sums, W1, b1, W2, b2)


# speedup vs baseline: 7.5507x; 1.0001x over previous
import functools

import jax
import jax.numpy as jnp
from jax import lax
from jax.experimental import pallas as pl
from jax.experimental.pallas import tpu as pltpu
from jax.experimental.pallas import tpu_sc as plsc

_NC = 2
_NS = 16
_NW = _NC * _NS

_SEQ = 200
_S0 = 96
_S1 = 104
_TBLK = 38400


def _stage_body(tt_ref, o_ref):
    h = _TBLK // 2
    o_ref[:, 0:64] = jnp.transpose(tt_ref[:, 0:h], (1, 0))
    o_ref[:, 64:128] = jnp.transpose(tt_ref[:, h:_TBLK], (1, 0))


def _stage_table(tt, vocab):
    return pl.pallas_call(
        _stage_body,
        grid=(pl.cdiv(vocab, _TBLK),),
        in_specs=[pl.BlockSpec((64, _TBLK), lambda i: (0, i))],
        out_specs=pl.BlockSpec((_TBLK // 2, 128), lambda i: (i, 0)),
        out_shape=jax.ShapeDtypeStruct(
            (pl.cdiv(vocab, _TBLK) * (_TBLK // 2), 128), jnp.float32),
    )(tt)


def _pool_sc(x, tab, batch):
    b_per_w = batch // _NW

    mesh = plsc.VectorSubcoreMesh(core_axis_name="c", subcore_axis_name="s")

    @functools.partial(
        pl.kernel,
        out_type=jax.ShapeDtypeStruct((batch, 64), jnp.float32),
        mesh=mesh,
        scratch_types=[
            pltpu.VMEM((b_per_w, _SEQ), jnp.int32),
            pltpu.VMEM((4, _SEQ, 64), jnp.float32),
            pltpu.VMEM((b_per_w, 64), jnp.float32),
            pltpu.SemaphoreType.DMA,
            pltpu.SemaphoreType.DMA,
            pltpu.SemaphoreType.DMA,
            pltpu.SemaphoreType.DMA,
        ],
        compiler_params=pltpu.CompilerParams(use_tc_tiling_on_sc=False),
    )
    def pool(x_hbm, tab_hbm, out_hbm, idx_v, rows_v, acc_v,
             sem0, sem1, sem2, sem3):
        sems = (sem0, sem1, sem2, sem3)
        wid = lax.axis_index("s") * _NC + lax.axis_index("c")
        pltpu.sync_copy(x_hbm.at[pl.ds(wid * b_per_w, b_per_w)], idx_v)

        def issue(r, buf, sem):
            pltpu.async_copy(
                tab_hbm.at[idx_v.at[r, pl.ds(0, _S0)]],
                rows_v.at[buf, pl.ds(0, _S0)], sem)
            pltpu.async_copy(
                tab_hbm.at[idx_v.at[r, pl.ds(_S0, _S1)]],
                rows_v.at[buf, pl.ds(_S0, _S1)], sem)

        def wait(buf, sem):
            pltpu.make_async_copy(
                tab_hbm.at[pl.ds(0, _SEQ)], rows_v.at[buf], sem).wait()

        def accumulate(r, buf):
            def body(s, carry):
                a0, a1, a2, a3 = carry
                a0 = a0 + rows_v[buf, s, pl.ds(0, 16)]
                a1 = a1 + rows_v[buf, s, pl.ds(16, 16)]
                a2 = a2 + rows_v[buf, s, pl.ds(32, 16)]
                a3 = a3 + rows_v[buf, s, pl.ds(48, 16)]
                return a0, a1, a2, a3

            z = jnp.zeros((16,), jnp.float32)
            a0, a1, a2, a3 = lax.fori_loop(0, _SEQ, body, (z, z, z, z),
                                           unroll=8)
            acc_v[r, pl.ds(0, 16)] = a0
            acc_v[r, pl.ds(16, 16)] = a1
            acc_v[r, pl.ds(32, 16)] = a2
            acc_v[r, pl.ds(48, 16)] = a3

        issue(0, 0, sem0)
        issue(1, 1, sem1)
        issue(2, 2, sem2)

        def quad_body(g, _):
            r0 = 4 * g
            for k in range(4):
                nxt = r0 + k + 3

                @pl.when(nxt < b_per_w)
                def _issue_next():
                    issue(nxt, (k + 3) % 4, sems[(k + 3) % 4])

                wait(k, sems[k])
                accumulate(r0 + k, k)
            return _

        lax.fori_loop(0, b_per_w // 4, quad_body, None)
        pltpu.sync_copy(acc_v, out_hbm.at[pl.ds(wid * b_per_w, b_per_w)])

    return pool(x, tab)


def _mlp_body(p_ref, w1_ref, b1_ref, w2_ref, b2_ref, o_ref):
    p = p_ref[...] * (1.0 / _SEQ)
    h = jnp.dot(p, w1_ref[...], preferred_element_type=jnp.float32) + b1_ref[...]
    h = jnp.maximum(h, 0.0)
    logits = jnp.dot(h, w2_ref[...], preferred_element_type=jnp.float32) + b2_ref[...]
    m = jnp.max(logits, axis=1, keepdims=True)
    ex = jnp.exp(logits - m)
    o_ref[...] = logits - m - jnp.log(jnp.sum(ex, axis=1, keepdims=True))


def _mlp_tc(sums, W1, b1, W2, b2):
    batch, embed = sums.shape
    hidden = W1.shape[1]
    out = W2.shape[1]
    blk = 512
    return pl.pallas_call(
        _mlp_body,
        grid=(batch // blk,),
        in_specs=[
            pl.BlockSpec((blk, embed), lambda i: (i, 0)),
            pl.BlockSpec((embed, hidden), lambda i: (0, 0)),
            pl.BlockSpec((1, hidden), lambda i: (0, 0)),
            pl.BlockSpec((hidden, out), lambda i: (0, 0)),
            pl.BlockSpec((1, out), lambda i: (0, 0)),
        ],
        out_specs=pl.BlockSpec((blk, out), lambda i: (i, 0)),
        out_shape=jax.ShapeDtypeStruct((batch, out), jnp.float32),
    )(sums, W1, b1.reshape(1, hidden), W2, b2.reshape(1, out))


def kernel(x, table, W1, b1, W2, b2):
    batch, seq = x.shape
    vocab, embed = table.shape
    assert seq == _SEQ and batch % _NW == 0 and embed == 64
    nrow = 2 * pl.cdiv(vocab, _TBLK) * (_TBLK // 2)
    tab = _stage_table(table.T, vocab).reshape(nrow, 64)
    u = x % _TBLK
    xf = x - u + 2 * (u % (_TBLK // 2)) + u // (_TBLK // 2)
    sums = _pool_sc(xf, tab, batch)
    return _mlp_tc(sums, W1, b1, W2, b2)
